# baseline (speedup 1.0000x reference)
import jax
import jax.numpy as jnp
from jax.experimental import pallas as pl


_BR = 1536
_SLICE_ROWS = 4096


def _body(x_ref, o1_ref, o2_ref):
    i = pl.program_id(0)
    v = x_ref[...]
    o1_ref[...] = v
    row = i * _BR + jax.lax.broadcasted_iota(jnp.int32, v.shape, 0)
    o2_ref[...] = jnp.where(row < _SLICE_ROWS, jnp.float32(2.0), v)


def kernel(x):
    n, r, c = x.shape
    xf = x.reshape(n * r, c)
    grid = ((n * r) // _BR,)
    spec = pl.BlockSpec((_BR, c), lambda i: (i, 0))
    out1, out2 = pl.pallas_call(
        _body,
        grid=grid,
        in_specs=[spec],
        out_specs=[spec, spec],
        out_shape=[
            jax.ShapeDtypeStruct(xf.shape, x.dtype),
            jax.ShapeDtypeStruct(xf.shape, x.dtype),
        ],
    )(xf)
    return (out1.reshape(x.shape), out2.reshape(x.shape))


# speedup vs baseline: 1.0152x; 1.0152x over previous
import jax
import jax.numpy as jnp
from jax.experimental import pallas as pl


_BR = 2048
_SLICE_ROWS = 4096


def _body(x_ref, o1_ref, o2_ref):
    i = pl.program_id(0)
    v = x_ref[...]
    o1_ref[...] = v
    row = i * _BR + jax.lax.broadcasted_iota(jnp.int32, v.shape, 0)
    o2_ref[...] = jnp.where(row < _SLICE_ROWS, jnp.float32(2.0), v)


def kernel(x):
    n, r, c = x.shape
    xf = x.reshape(n * r, c)
    grid = ((n * r) // _BR,)
    spec = pl.BlockSpec((_BR, c), lambda i: (i, 0))
    out1, out2 = pl.pallas_call(
        _body,
        grid=grid,
        in_specs=[spec],
        out_specs=[spec, spec],
        out_shape=[
            jax.ShapeDtypeStruct(xf.shape, x.dtype),
            jax.ShapeDtypeStruct(xf.shape, x.dtype),
        ],
    )(xf)
    return (out1.reshape(x.shape), out2.reshape(x.shape))
